# Initial kernel scaffold; baseline (speedup 1.0000x reference)
import jax
import jax.numpy as jnp
from jax.experimental import pallas as pl


def kernel(g_i, mention_scores, mention_ids, start_indices, end_indices, genre_ids, speaker_ids, dist_tab, dist_coarse_tab, genre_tab, speaker_tab, coarse_W, dist_proj_W, dist_proj_b, score_W1, score_b1, score_W2, score_b2, Wf_W, Wf_b):
    raise NotImplementedError("write your pallas kernel here")


# speedup vs baseline: 5.4434x; 5.4434x over previous
import functools

import jax
import jax.numpy as jnp
from jax.experimental import pallas as pl

K = 8
NEG = -1e30
BINS = (1, 2, 3, 4, 8, 16, 32, 64)

f32 = jnp.float32
bf16 = jnp.bfloat16
i32 = jnp.int32


def _dot(a, b):
    return jax.lax.dot_general(a.astype(bf16), b.astype(bf16),
                               (((1,), (0,)), ((), ())),
                               preferred_element_type=f32)


def _dot_t(a, b):
    return jax.lax.dot_general(a.astype(bf16), b.astype(bf16),
                               (((1,), (1,)), ((), ())),
                               preferred_element_type=f32)


def _iota(shape, dim):
    return jax.lax.broadcasted_iota(i32, shape, dim)


def _gather_body(g_ref, ms_row_ref, mid_blk_ref, mid_row_ref,
                 ig_ref, si_ref, rep_ref):
    mid_blk = mid_blk_ref[...]
    B = mid_blk.shape[0]
    oh = mid_blk == _iota((B, g_ref.shape[0]), 1)
    ig_ref[...] = _dot(oh.astype(bf16), g_ref[...])
    ms_row = ms_row_ref[...]
    si_ref[...] = jnp.sum(jnp.where(oh, ms_row, 0.0), axis=1, keepdims=True)
    mid_row = mid_row_ref[...]
    rep_ref[...] = jnp.sum((mid_row <= mid_blk).astype(i32), axis=1,
                           keepdims=True) - 1


def _coarse_body(igb_ref, igf_ref, cw_ref, sib_ref, sir_ref,
                 ivals_ref, end_ref, genre_ref, spk_ref,
                 dist_ref, genre_tab_ref, spk_tab_ref,
                 bs_ref, bi_ref, repbi_ref, phi_ref):
    B = igb_ref.shape[0]
    k = igf_ref.shape[0]
    pid = pl.program_id(0)
    a = _dot_t(igb_ref[...], cw_ref[...])
    ant = _dot_t(a, igf_ref[...])
    rowg = pid * B + _iota((B, 1), 0)
    col = _iota((B, k), 1)
    ant = ant + sib_ref[...] + sir_ref[...]
    ant = ant + jnp.where((rowg - col) >= 1, 0.0, NEG)

    bs_cols, bi_cols = [], []
    for _ in range(K):
        m = jnp.max(ant, axis=1, keepdims=True)
        ismax = ant == m
        idx = jnp.min(jnp.where(ismax, col, k + 1), axis=1, keepdims=True)
        bs_cols.append(m)
        bi_cols.append(idx)
        ant = jnp.where(col == idx, -3.0e38, ant)
    bs_ref[...] = jnp.concatenate(bs_cols, axis=1)
    bi = jnp.concatenate(bi_cols, axis=1)
    bi_ref[...] = bi

    ivals = ivals_ref[...].astype(bf16)
    end_b = end_ref[...]
    genre_b = genre_ref[...]
    spk_b = spk_ref[...]
    ohg = (genre_b == _iota((B, genre_tab_ref.shape[0]), 1))
    phi_g = _dot(ohg.astype(bf16), genre_tab_ref[...])
    s1 = spk_tab_ref[1:2, :]
    s2 = spk_tab_ref[2:3, :]

    rb_cols = []
    for j in range(K):
        ohj = (bi[:, j:j + 1] == _iota((B, k), 1)).astype(bf16)
        g5 = _dot(ohj, ivals)
        start_j = g5[:, 0:1] + 128.0 * g5[:, 1:2]
        spk_j = g5[:, 2:3]
        rep_j = g5[:, 3:4] + 128.0 * g5[:, 4:5]
        rb_cols.append(rep_j.astype(i32))
        dfine = end_b - start_j.astype(i32)
        bucket = jnp.zeros_like(dfine)
        for b in BINS:
            bucket = bucket + (dfine > b).astype(i32)
        ohd = (bucket == _iota((B, dist_ref.shape[0]), 1)).astype(bf16)
        phi_d = _dot(ohd, dist_ref[...])
        sp2 = spk_j.astype(i32) != spk_b
        phi_s = jnp.where(sp2, s2, s1)
        pj = jnp.concatenate(
            [phi_d, phi_g, phi_s, jnp.zeros((B, 4), f32)], axis=1)
        phi_ref[j, :, :] = pj
    repbi_ref[...] = jnp.concatenate(rb_cols, axis=1)


def _mlp_body(step, igb_ref, src_ref, rep_ref, repbi_ref, bi_ref, bs_ref,
              phi_ref, w1i_ref, w1j_ref, w1p_ref, w1f_ref, b1_ref, w2_ref,
              b2_ref, wfi_ref, wfa_ref, wfb_ref, out_ref):
    B = bi_ref.shape[0]
    k = src_ref.shape[0]
    pid = pl.program_id(0)
    rowg = pid * B + _iota((B, 1), 0)
    src_b = src_ref[...].astype(bf16)
    if step == 0:
        ig = igb_ref[...]
    else:
        ohr = (rep_ref[...] == _iota((B, k), 1)).astype(bf16)
        ig = _dot(ohr, src_b)
    hi = _dot(ig, w1i_ref[...])
    b1 = b1_ref[...]
    w2 = w2_ref[...]
    bi = bi_ref[...]
    if step == 0:
        jdx = bi
    else:
        jdx = repbi_ref[...]
    bs = bs_ref[...]
    jgs, cs = [], []
    for j in range(K):
        ohj = (jdx[:, j:j + 1] == _iota((B, k), 1)).astype(bf16)
        jg = _dot(ohj, src_b)
        h = hi + _dot(jg, w1j_ref[...]) + _dot(ig * jg, w1p_ref[...]) \
            + _dot(phi_ref[j, :, :], w1f_ref[...]) + b1
        h = jnp.maximum(h, 0.0)
        s = jnp.sum(h * w2, axis=1, keepdims=True) + b2_ref[...]
        c = jnp.where(bi[:, j:j + 1] < rowg, s + bs[:, j:j + 1], -1e10)
        jgs.append(jg)
        cs.append(c)
    if step == 1:
        out_ref[...] = jnp.concatenate(
            [jnp.zeros((B, 1), f32)] + cs, axis=1)
        return
    m = jnp.zeros((B, 1), f32)
    for c in cs:
        m = jnp.maximum(m, c)
    z = jnp.exp(-m)
    es = []
    for c in cs:
        e = jnp.exp(c - m)
        es.append(e)
        z = z + e
    a_n = ig * (jnp.exp(-m) / z)
    for e, jg in zip(es, jgs):
        a_n = a_n + jg * (e / z)
    f_pre = _dot_t(ig, wfi_ref[...]) + _dot_t(a_n, wfa_ref[...]) + wfb_ref[...]
    f_n = jax.nn.sigmoid(f_pre)
    out_ref[...] = f_n * ig + (1.0 - f_n) * a_n


def _full(shape):
    n = len(shape)
    return pl.BlockSpec(shape, lambda i, _n=n: (0,) * _n)


def _rows(bshape):
    return pl.BlockSpec(bshape, lambda i: (i,) + (0,) * (len(bshape) - 1))


def kernel(g_i, mention_scores, mention_ids, start_indices, end_indices,
           genre_ids, speaker_ids, dist_tab, dist_coarse_tab, genre_tab,
           speaker_tab, coarse_W, dist_proj_W, dist_proj_b, score_W1,
           score_b1, score_W2, score_b2, Wf_W, Wf_b):
    M, d = g_i.shape
    k = mention_ids.shape[0]
    B = 256
    G = k // B
    H = score_W1.shape[0]

    mid_col = mention_ids.reshape(k, 1).astype(i32)
    mid_row = mention_ids.reshape(1, k).astype(i32)
    ms_row = mention_scores.reshape(1, M)

    ig, si, rep = pl.pallas_call(
        _gather_body,
        grid=(G,),
        in_specs=[_full((M, d)), _full((1, M)), _rows((B, 1)), _full((1, k))],
        out_specs=[_rows((B, d)), _rows((B, 1)), _rows((B, 1))],
        out_shape=[jax.ShapeDtypeStruct((k, d), f32),
                   jax.ShapeDtypeStruct((k, 1), f32),
                   jax.ShapeDtypeStruct((k, 1), i32)],
    )(g_i, ms_row, mid_col, mid_row)

    start_col = start_indices.reshape(k, 1).astype(i32)
    end_col = end_indices.reshape(k, 1).astype(i32)
    genre_col = genre_ids.reshape(k, 1).astype(i32)
    spk_col = speaker_ids.reshape(k, 1).astype(i32)
    ivals = jnp.concatenate(
        [start_col & 127, start_col >> 7, spk_col, rep & 127, rep >> 7],
        axis=1).astype(i32)
    dist_pad = jnp.zeros((16, 20), f32).at[:9].set(dist_tab)
    spk_pad = jnp.zeros((8, 20), f32).at[:3].set(speaker_tab)

    bs, bi, repbi, phi = pl.pallas_call(
        _coarse_body,
        grid=(G,),
        in_specs=[_rows((B, d)), _full((k, d)), _full((d, d)),
                  _rows((B, 1)), _full((1, k)), _full((k, 5)),
                  _rows((B, 1)), _rows((B, 1)), _rows((B, 1)),
                  _full((16, 20)), _full((8, 20)), _full((8, 20))],
        out_specs=[_rows((B, K)), _rows((B, K)), _rows((B, K)),
                   pl.BlockSpec((K, B, 64), lambda i: (0, i, 0))],
        out_shape=[jax.ShapeDtypeStruct((k, K), f32),
                   jax.ShapeDtypeStruct((k, K), i32),
                   jax.ShapeDtypeStruct((k, K), i32),
                   jax.ShapeDtypeStruct((K, k, 64), f32)],
    )(ig, ig, coarse_W, si, si.reshape(1, k), ivals, end_col, genre_col,
      spk_col, dist_pad, genre_tab, spk_pad)

    w1i = score_W1[:, :d].T
    w1j = score_W1[:, d:2 * d].T
    w1p = score_W1[:, 2 * d:3 * d].T
    w1f = jnp.zeros((64, H), f32).at[:60].set(score_W1[:, 3 * d:].T)
    b1_row = score_b1.reshape(1, H)
    w2_row = score_W2.reshape(1, H)
    b2_11 = score_b2.reshape(1, 1)
    wfi = Wf_W[:, :d]
    wfa = Wf_W[:, d:]
    wfb_row = Wf_b.reshape(1, d)

    mlp_specs = [_rows((B, d)), _full((k, d)), _rows((B, 1)), _rows((B, K)),
                 _rows((B, K)), _rows((B, K)),
                 pl.BlockSpec((K, B, 64), lambda i: (0, i, 0)),
                 _full((d, H)), _full((d, H)), _full((d, H)), _full((64, H)),
                 _full((1, H)), _full((1, H)), _full((1, 1)),
                 _full((d, d)), _full((d, d)), _full((1, d))]

    u = pl.pallas_call(
        functools.partial(_mlp_body, 0),
        grid=(G,),
        in_specs=mlp_specs,
        out_specs=_rows((B, d)),
        out_shape=jax.ShapeDtypeStruct((k, d), f32),
    )(ig, ig, rep, repbi, bi, bs, phi, w1i, w1j, w1p, w1f, b1_row, w2_row,
      b2_11, wfi, wfa, wfb_row)

    scores = pl.pallas_call(
        functools.partial(_mlp_body, 1),
        grid=(G,),
        in_specs=mlp_specs,
        out_specs=_rows((B, 1 + K)),
        out_shape=jax.ShapeDtypeStruct((k, 1 + K), f32),
    )(ig, u, rep, repbi, bi, bs, phi, w1i, w1j, w1p, w1f, b1_row, w2_row,
      b2_11, wfi, wfa, wfb_row)
    return scores
